# Initial kernel scaffold; baseline (speedup 1.0000x reference)
import jax
import jax.numpy as jnp
from jax.experimental import pallas as pl


def kernel(x, edge_index, pos_cat, params):
    raise NotImplementedError("write your pallas kernel here")


# speedup vs baseline: 2.4991x; 2.4991x over previous
import functools

import jax
import jax.numpy as jnp
from jax import lax
from jax.experimental import pallas as pl
from jax.experimental.pallas import tpu as pltpu
from jax.experimental.pallas import tpu_sc as plsc

N = 10000
E = 320000
D = 128
H = 128

NC = 2
NS = 16
NW = NC * NS
EPW = E // NW
CHUNK = 128
NFULL = EPW // CHUNK
LASTOFF = EPW - CHUNK
OVERLAP = NFULL * CHUNK - LASTOFF
NCH = NFULL + 1
TRASH = N
NP8 = N + 8

_mesh = functools.partial(
    plsc.VectorSubcoreMesh, core_axis_name="c", subcore_axis_name="s")


def _chunk_off(base, i):
    return base + jnp.minimum(i * CHUNK, LASTOFF)


def _sc_dist_deg(px, py, pz, idx_dst, idx_src, ones_rows, zeros128):
    @functools.partial(
        pl.kernel,
        out_type=(
            jax.ShapeDtypeStruct((E,), jnp.float32),
            jax.ShapeDtypeStruct((NC, NP8, H), jnp.float32),
        ),
        mesh=_mesh(),
        scratch_types=[
            pltpu.VMEM((CHUNK,), jnp.int32),
            pltpu.VMEM((CHUNK,), jnp.int32),
            pltpu.VMEM((CHUNK,), jnp.float32),
            pltpu.VMEM((CHUNK,), jnp.float32),
            pltpu.VMEM((CHUNK,), jnp.float32),
            pltpu.VMEM((CHUNK,), jnp.float32),
            pltpu.VMEM((CHUNK,), jnp.float32),
            pltpu.VMEM((CHUNK,), jnp.float32),
            pltpu.VMEM((CHUNK,), jnp.float32),
            pltpu.VMEM((CHUNK, H), jnp.float32),
            pltpu.VMEM_SHARED((NP8, H), jnp.float32),
            pltpu.SemaphoreType.DMA,
        ],
    )
    def k(px_hbm, py_hbm, pz_hbm, dst_hbm, src_hbm, ones_hbm, zeros_hbm,
          out_d2, out_deg, id_v, is_v, pdx_v, pdy_v, pdz_v,
          psx_v, psy_v, psz_v, d2_v, ones_v, acc, sem):
        c = lax.axis_index("c")
        s = lax.axis_index("s")
        wid = c * NS + s
        base = wid * EPW

        @pl.when(s == 0)
        def _():
            pltpu.sync_copy(zeros_hbm, acc)
        pltpu.sync_copy(ones_hbm, ones_v)
        plsc.subcore_barrier()

        def body(i, carry):
            off = _chunk_off(base, i)
            pltpu.sync_copy(dst_hbm.at[pl.ds(off, CHUNK)], id_v)
            pltpu.sync_copy(src_hbm.at[pl.ds(off, CHUNK)], is_v)
            pltpu.async_copy(px_hbm.at[id_v], pdx_v, sem).wait()
            pltpu.async_copy(py_hbm.at[id_v], pdy_v, sem).wait()
            pltpu.async_copy(pz_hbm.at[id_v], pdz_v, sem).wait()
            pltpu.async_copy(px_hbm.at[is_v], psx_v, sem).wait()
            pltpu.async_copy(py_hbm.at[is_v], psy_v, sem).wait()
            pltpu.async_copy(pz_hbm.at[is_v], psz_v, sem).wait()
            for j in range(CHUNK // 16):
                sl = pl.ds(j * 16, 16)
                rx = pdx_v[sl] - psx_v[sl]
                ry = pdy_v[sl] - psy_v[sl]
                rz = pdz_v[sl] - psz_v[sl]
                d2_v[sl] = rx * rx + ry * ry + rz * rz
            pltpu.sync_copy(d2_v, out_d2.at[pl.ds(off, CHUNK)])

            @pl.when(i == NFULL)
            def _():
                for j in range(OVERLAP // 16):
                    id_v[pl.ds(j * 16, 16)] = jnp.full((16,), TRASH, jnp.int32)
            pltpu.sync_copy(ones_v, acc.at[id_v], add=True)
            return carry

        lax.fori_loop(0, NCH, body, 0)
        plsc.subcore_barrier()

        @pl.when(s == 0)
        def _():
            pltpu.sync_copy(acc, out_deg.at[c])

    return k(px, py, pz, idx_dst, idx_src, ones_rows, zeros128)


def _sc_gather2(tab_a, tab_b, idx_a, idx_b):
    @functools.partial(
        pl.kernel,
        out_type=(
            jax.ShapeDtypeStruct((E, H), jnp.float32),
            jax.ShapeDtypeStruct((E, H), jnp.float32),
        ),
        mesh=_mesh(),
        scratch_types=[
            pltpu.VMEM((CHUNK,), jnp.int32),
            pltpu.VMEM((CHUNK,), jnp.int32),
            pltpu.VMEM((CHUNK, H), jnp.float32),
            pltpu.VMEM((CHUNK, H), jnp.float32),
            pltpu.SemaphoreType.DMA,
        ],
    )
    def k(ta_hbm, tb_hbm, ia_hbm, ib_hbm, out_a, out_b,
          ia_v, ib_v, ra_v, rb_v, sem):
        c = lax.axis_index("c")
        s = lax.axis_index("s")
        base = (c * NS + s) * EPW

        def body(i, carry):
            off = _chunk_off(base, i)
            pltpu.sync_copy(ia_hbm.at[pl.ds(off, CHUNK)], ia_v)
            pltpu.sync_copy(ib_hbm.at[pl.ds(off, CHUNK)], ib_v)
            pltpu.async_copy(ta_hbm.at[ia_v], ra_v, sem).wait()
            pltpu.async_copy(tb_hbm.at[ib_v], rb_v, sem).wait()
            pltpu.sync_copy(ra_v, out_a.at[pl.ds(off, CHUNK)])
            pltpu.sync_copy(rb_v, out_b.at[pl.ds(off, CHUNK)])
            return carry

        lax.fori_loop(0, NCH, body, 0)

    return k(tab_a, tab_b, idx_a, idx_b)


def _sc_scatter(g, idx_dst, zeros128):
    @functools.partial(
        pl.kernel,
        out_type=jax.ShapeDtypeStruct((NC, NP8, H), jnp.float32),
        mesh=_mesh(),
        scratch_types=[
            pltpu.VMEM((CHUNK,), jnp.int32),
            pltpu.VMEM((CHUNK, H), jnp.float32),
            pltpu.VMEM_SHARED((NP8, H), jnp.float32),
            pltpu.SemaphoreType.DMA,
        ],
    )
    def k(g_hbm, dst_hbm, zeros_hbm, out, id_v, g_v, acc, sem):
        c = lax.axis_index("c")
        s = lax.axis_index("s")
        base = (c * NS + s) * EPW

        @pl.when(s == 0)
        def _():
            pltpu.sync_copy(zeros_hbm, acc)
        plsc.subcore_barrier()

        def body(i, carry):
            off = _chunk_off(base, i)
            pltpu.sync_copy(dst_hbm.at[pl.ds(off, CHUNK)], id_v)
            pltpu.sync_copy(g_hbm.at[pl.ds(off, CHUNK)], g_v)

            @pl.when(i == NFULL)
            def _():
                for j in range(OVERLAP // 16):
                    id_v[pl.ds(j * 16, 16)] = jnp.full((16,), TRASH, jnp.int32)
            pltpu.sync_copy(g_v, acc.at[id_v], add=True)
            return carry

        lax.fori_loop(0, NCH, body, 0)
        plsc.subcore_barrier()

        @pl.when(s == 0)
        def _():
            pltpu.sync_copy(acc, out.at[c])

    return k(g, idx_dst, zeros128)


_NBLK = 400
_EBLK = 512


def _tc_prep(x, posp, w1a, w1b, cpad):
    def body(x_ref, p_ref, wa_ref, wb_ref, c_ref, u_ref, v_ref):
        xb = x_ref[...]
        wb = wb_ref[...]
        pc = jnp.dot(p_ref[...], c_ref[...], preferred_element_type=jnp.float32)
        u_ref[...] = jnp.dot(xb, wa_ref[...] - wb,
                             preferred_element_type=jnp.float32) + pc
        v_ref[...] = jnp.dot(xb, wb, preferred_element_type=jnp.float32) - pc

    return pl.pallas_call(
        body,
        grid=(N // _NBLK,),
        in_specs=[
            pl.BlockSpec((_NBLK, D), lambda i: (i, 0)),
            pl.BlockSpec((_NBLK, H), lambda i: (i, 0)),
            pl.BlockSpec((D, H), lambda i: (0, 0)),
            pl.BlockSpec((D, H), lambda i: (0, 0)),
            pl.BlockSpec((H, H), lambda i: (0, 0)),
        ],
        out_specs=[
            pl.BlockSpec((_NBLK, H), lambda i: (i, 0)),
            pl.BlockSpec((_NBLK, H), lambda i: (i, 0)),
        ],
        out_shape=[
            jax.ShapeDtypeStruct((N, H), jnp.float32),
            jax.ShapeDtypeStruct((N, H), jnp.float32),
        ],
    )(x, posp, w1a, w1b, cpad)


def _tc_gelu(d2r, ud, vs, c3, b1):
    def body(d2_ref, ud_ref, vs_ref, c3_ref, b1_ref, g_ref):
        dt = jnp.transpose(d2_ref[0])
        dist = jnp.sqrt(jnp.concatenate(
            [dt[:, r:r + 1] for r in range(_EBLK // 128)], axis=0))
        pre = ud_ref[...] + vs_ref[...] + dist * c3_ref[...] + b1_ref[...]
        g_ref[...] = 0.5 * pre * (1.0 + lax.erf(pre * (2.0 ** -0.5)))

    return pl.pallas_call(
        body,
        grid=(E // _EBLK,),
        in_specs=[
            pl.BlockSpec((1, _EBLK // 128, 128), lambda i: (i, 0, 0)),
            pl.BlockSpec((_EBLK, H), lambda i: (i, 0)),
            pl.BlockSpec((_EBLK, H), lambda i: (i, 0)),
            pl.BlockSpec((1, H), lambda i: (0, 0)),
            pl.BlockSpec((1, H), lambda i: (0, 0)),
        ],
        out_specs=pl.BlockSpec((_EBLK, H), lambda i: (i, 0)),
        out_shape=jax.ShapeDtypeStruct((E, H), jnp.float32),
    )(d2r, ud, vs, c3, b1)


def _layer_norm(y, g, b):
    m = jnp.mean(y, axis=-1, keepdims=True)
    v = jnp.mean((y - m) ** 2, axis=-1, keepdims=True)
    return (y - m) * lax.rsqrt(v + 1e-5) * g + b


def _tc_finalize(part, degpart, w2, b2, ln_g, ln_b, xres, add_res):
    def body(p_ref, d_ref, w2_ref, b2_ref, g_ref, bb_ref, x_ref, o_ref):
        p = p_ref[0] + p_ref[1]
        deg = d_ref[0, :, 0:1] + d_ref[1, :, 0:1]
        agg = (jnp.dot(p, w2_ref[...], preferred_element_type=jnp.float32)
               + deg * b2_ref[...])
        out = _layer_norm(agg, g_ref[...], bb_ref[...])
        if add_res:
            out = out + x_ref[...]
        o_ref[...] = out

    return pl.pallas_call(
        body,
        grid=(N // _NBLK,),
        in_specs=[
            pl.BlockSpec((NC, _NBLK, H), lambda i: (0, i, 0)),
            pl.BlockSpec((NC, _NBLK, H), lambda i: (0, i, 0)),
            pl.BlockSpec((H, H), lambda i: (0, 0)),
            pl.BlockSpec((1, H), lambda i: (0, 0)),
            pl.BlockSpec((1, H), lambda i: (0, 0)),
            pl.BlockSpec((1, H), lambda i: (0, 0)),
            pl.BlockSpec((_NBLK, H), lambda i: (i, 0)),
        ],
        out_specs=pl.BlockSpec((_NBLK, H), lambda i: (i, 0)),
        out_shape=jax.ShapeDtypeStruct((N, H), jnp.float32),
    )(part, degpart, w2, b2, ln_g, ln_b, xres)


def _tc_head(x, ln_g, ln_b, hw1, hb1, hw2p, hb2p):
    def body(x_ref, g_ref, b_ref, w1_ref, b1_ref, w2_ref, b2_ref, o_ref):
        y = _layer_norm(x_ref[...], g_ref[...], b_ref[...])
        t = jnp.dot(y, w1_ref[...], preferred_element_type=jnp.float32) \
            + b1_ref[...]
        t = 0.5 * t * (1.0 + lax.erf(t * (2.0 ** -0.5)))
        o_ref[...] = jnp.dot(t, w2_ref[...],
                             preferred_element_type=jnp.float32) + b2_ref[...]

    return pl.pallas_call(
        body,
        grid=(N // _NBLK,),
        in_specs=[
            pl.BlockSpec((_NBLK, H), lambda i: (i, 0)),
            pl.BlockSpec((1, H), lambda i: (0, 0)),
            pl.BlockSpec((1, H), lambda i: (0, 0)),
            pl.BlockSpec((H, H), lambda i: (0, 0)),
            pl.BlockSpec((1, H), lambda i: (0, 0)),
            pl.BlockSpec((H, H), lambda i: (0, 0)),
            pl.BlockSpec((1, H), lambda i: (0, 0)),
        ],
        out_specs=pl.BlockSpec((_NBLK, H), lambda i: (i, 0)),
        out_shape=jax.ShapeDtypeStruct((N, H), jnp.float32),
    )(x, ln_g, ln_b, hw1, hb1, hw2p, hb2p)


def kernel(x, edge_index, pos_cat, params):
    src = edge_index[0]
    dst = edge_index[1]

    px, py, pz = pos_cat[:, 0], pos_cat[:, 1], pos_cat[:, 2]
    posp = jnp.zeros((N, H), jnp.float32).at[:, :3].set(pos_cat)
    ones_rows = jnp.ones((CHUNK, H), jnp.float32)
    zeros128 = jnp.zeros((NP8, H), jnp.float32)

    d2, degpart = _sc_dist_deg(px, py, pz, dst, src, ones_rows, zeros128)
    d2r = d2.reshape(E // _EBLK, _EBLK // 128, 128)

    x_cur = x
    for i, p in enumerate(params["convs"]):
        w1a = p["W1"][0:H]
        w1b = p["W1"][H:2 * H]
        cpad = jnp.zeros((H, H), jnp.float32).at[0:3].set(p["W1"][2 * H:2 * H + 3])
        c3 = p["W1"][2 * H + 3:].reshape(1, H)
        b1 = p["b1"].reshape(1, H)
        u, v = _tc_prep(x_cur, posp, w1a, w1b, cpad)
        ud, vs = _sc_gather2(u, v, dst, src)
        g = _tc_gelu(d2r, ud, vs, c3, b1)
        part = _sc_scatter(g, dst, zeros128)
        x_cur = _tc_finalize(part, degpart, p["W2"], p["b2"].reshape(1, H),
                             p["ln_g"].reshape(1, H), p["ln_b"].reshape(1, H),
                             x_cur, add_res=(i > 0))

    h = params["head"]
    hw2p = jnp.zeros((H, H), jnp.float32).at[:, :3].set(h["W2"])
    hb2p = jnp.zeros((1, H), jnp.float32).at[0, :3].set(h["b2"])
    y = _tc_head(x_cur, h["ln_g"].reshape(1, H), h["ln_b"].reshape(1, H),
                 h["W1"], h["b1"].reshape(1, H), hw2p, hb2p)
    return y[:, :3]
